# baseline (speedup 1.0000x reference)
import jax
import jax.numpy as jnp
from jax import lax
from jax.experimental import pallas as pl

_K = 64
_ROWS_PER_BLOCK = 16
_N = 32768
_NG = 2048
_IMIN = jnp.iinfo(jnp.int32).min


def _lane_cumsum(y, width):
    s = 1
    while s < width:
        shifted = jnp.concatenate(
            [jnp.zeros(y.shape[:-1] + (s,), y.dtype), y[..., :-s]], axis=-1
        )
        y = y + shifted
        s *= 2
    return y


def _topk_body(x_ref, o_ref):
    xb = x_ref[...]
    u = lax.bitcast_convert_type(xb, jnp.int32)
    key = u ^ ((u >> 31) & jnp.int32(0x7FFFFFFF))
    rows = xb.shape[0]

    mk = key[:, 0:_NG]
    for c in range(1, 16):
        mk = jnp.maximum(mk, key[:, c * _NG:(c + 1) * _NG])

    def gstep(i, t):
        trial = t + (jnp.int32(1) << (jnp.int32(31) - i))
        cnt = jnp.sum((mk >= trial).astype(jnp.int32), axis=1, keepdims=True)
        return jnp.where(cnt >= _K, trial, t)

    t0 = jnp.full((rows, 1), _IMIN, jnp.int32)
    l = lax.fori_loop(0, 32, gstep, t0)
    rmax = jnp.max(mk, axis=1, keepdims=True)

    d = l ^ rmax
    dp = jnp.maximum(d, 1)
    e = (lax.bitcast_convert_type(dp.astype(jnp.float32), jnp.int32) >> 23) - 127
    e = jnp.clip(e, 0, 30)
    e = jnp.where((dp & (jnp.int32(1) << e)) != 0, e, e - 1)
    b = jnp.where(d < 0, jnp.int32(31), jnp.where(d == 0, jnp.int32(0), e))
    tstart = jnp.where(
        b == 31,
        jnp.full((rows, 1), _IMIN, jnp.int32),
        l & ~((jnp.int32(2) << b) - 1),
    )
    bmax = jnp.max(b)

    def step(i, t):
        hi = jnp.maximum(b - 2 * i, 0)
        lo = jnp.maximum(b - 2 * i - 1, 0)
        ta = t + (jnp.int32(1) << hi)
        tb = t + (jnp.int32(1) << lo)
        tab = ta + (jnp.int32(1) << lo)
        ca = jnp.sum((key >= ta).astype(jnp.int32), axis=1, keepdims=True)
        cb = jnp.sum((key >= tb).astype(jnp.int32), axis=1, keepdims=True)
        cab = jnp.sum((key >= tab).astype(jnp.int32), axis=1, keepdims=True)
        return jnp.where(
            ca >= _K,
            jnp.where(cab >= _K, tab, ta),
            jnp.where(cb >= _K, tb, t),
        )

    nround = (bmax >> 1) + 1
    t = lax.fori_loop(0, nround, step, tstart)

    ge = key >= t
    n_ge = jnp.sum(ge.astype(jnp.int32), axis=1, keepdims=True)
    simple = jnp.all(n_ge == _K)

    @pl.when(simple)
    def _():
        o_ref[...] = jnp.maximum(jnp.where(ge, xb, 0.0), 0.0)

    @pl.when(jnp.logical_not(simple))
    def _():
        gt = key > t
        n_gt = jnp.sum(gt.astype(jnp.int32), axis=1, keepdims=True)
        r = _K - n_gt

        eq = (key == t).astype(jnp.int32)
        chunks = _N // 128
        e3 = eq.reshape(rows * chunks, 128)
        lane_incl = _lane_cumsum(e3, 128)
        chunk_tot = lane_incl[:, 127:128].reshape(rows, chunks)
        chunk_incl = _lane_cumsum(chunk_tot, chunks)
        chunk_excl = (chunk_incl - chunk_tot).reshape(rows * chunks, 1)
        prefix_excl = (chunk_excl + lane_incl - e3).reshape(rows, _N)

        keep = gt | ((eq > 0) & (prefix_excl < r))
        o_ref[...] = jnp.maximum(jnp.where(keep, xb, 0.0), 0.0)


@jax.jit
def kernel(x):
    m, n = x.shape
    grid = (m // _ROWS_PER_BLOCK,)
    return pl.pallas_call(
        _topk_body,
        grid=grid,
        in_specs=[pl.BlockSpec((_ROWS_PER_BLOCK, n), lambda i: (i, 0))],
        out_specs=pl.BlockSpec((_ROWS_PER_BLOCK, n), lambda i: (i, 0)),
        out_shape=jax.ShapeDtypeStruct((m, n), x.dtype),
    )(x)


# speedup vs baseline: 1.1683x; 1.1683x over previous
import jax
import jax.numpy as jnp
from jax import lax
from jax.experimental import pallas as pl

_K = 64
_ROWS_PER_BLOCK = 32
_N = 32768
_NG = 2048
_IMIN = jnp.iinfo(jnp.int32).min


def _lane_cumsum(y, width):
    s = 1
    while s < width:
        shifted = jnp.concatenate(
            [jnp.zeros(y.shape[:-1] + (s,), y.dtype), y[..., :-s]], axis=-1
        )
        y = y + shifted
        s *= 2
    return y


def _topk_body(x_ref, o_ref):
    xb = x_ref[...]
    u = lax.bitcast_convert_type(xb, jnp.int32)
    key = u ^ ((u >> 31) & jnp.int32(0x7FFFFFFF))
    rows = xb.shape[0]

    mk = key[:, 0:_NG]
    for c in range(1, 16):
        mk = jnp.maximum(mk, key[:, c * _NG:(c + 1) * _NG])

    def gstep(i, t):
        trial = t + (jnp.int32(1) << (jnp.int32(31) - i))
        cnt = jnp.sum((mk >= trial).astype(jnp.int32), axis=1, keepdims=True)
        return jnp.where(cnt >= _K, trial, t)

    t0 = jnp.full((rows, 1), _IMIN, jnp.int32)
    l = lax.fori_loop(0, 32, gstep, t0)
    rmax = jnp.max(mk, axis=1, keepdims=True)

    d = l ^ rmax
    dp = jnp.maximum(d, 1)
    e = (lax.bitcast_convert_type(dp.astype(jnp.float32), jnp.int32) >> 23) - 127
    e = jnp.clip(e, 0, 30)
    e = jnp.where((dp & (jnp.int32(1) << e)) != 0, e, e - 1)
    b = jnp.where(d < 0, jnp.int32(31), jnp.where(d == 0, jnp.int32(0), e))
    tstart = jnp.where(
        b == 31,
        jnp.full((rows, 1), _IMIN, jnp.int32),
        l & ~((jnp.int32(2) << b) - 1),
    )
    bmax = jnp.max(b)

    def step(i, t):
        hi = jnp.maximum(b - 2 * i, 0)
        lo = jnp.maximum(b - 2 * i - 1, 0)
        ta = t + (jnp.int32(1) << hi)
        tb = t + (jnp.int32(1) << lo)
        tab = ta + (jnp.int32(1) << lo)
        ca = jnp.sum((key >= ta).astype(jnp.int32), axis=1, keepdims=True)
        cb = jnp.sum((key >= tb).astype(jnp.int32), axis=1, keepdims=True)
        cab = jnp.sum((key >= tab).astype(jnp.int32), axis=1, keepdims=True)
        return jnp.where(
            ca >= _K,
            jnp.where(cab >= _K, tab, ta),
            jnp.where(cb >= _K, tb, t),
        )

    nround = (bmax >> 1) + 1
    t = lax.fori_loop(0, nround, step, tstart)

    ge = key >= t
    n_ge = jnp.sum(ge.astype(jnp.int32), axis=1, keepdims=True)
    simple = jnp.all(n_ge == _K)

    @pl.when(simple)
    def _():
        o_ref[...] = jnp.maximum(jnp.where(ge, xb, 0.0), 0.0)

    @pl.when(jnp.logical_not(simple))
    def _():
        gt = key > t
        n_gt = jnp.sum(gt.astype(jnp.int32), axis=1, keepdims=True)
        r = _K - n_gt

        eq = (key == t).astype(jnp.int32)
        chunks = _N // 128
        e3 = eq.reshape(rows * chunks, 128)
        lane_incl = _lane_cumsum(e3, 128)
        chunk_tot = lane_incl[:, 127:128].reshape(rows, chunks)
        chunk_incl = _lane_cumsum(chunk_tot, chunks)
        chunk_excl = (chunk_incl - chunk_tot).reshape(rows * chunks, 1)
        prefix_excl = (chunk_excl + lane_incl - e3).reshape(rows, _N)

        keep = gt | ((eq > 0) & (prefix_excl < r))
        o_ref[...] = jnp.maximum(jnp.where(keep, xb, 0.0), 0.0)


@jax.jit
def kernel(x):
    m, n = x.shape
    grid = (m // _ROWS_PER_BLOCK,)
    return pl.pallas_call(
        _topk_body,
        grid=grid,
        in_specs=[pl.BlockSpec((_ROWS_PER_BLOCK, n), lambda i: (i, 0))],
        out_specs=pl.BlockSpec((_ROWS_PER_BLOCK, n), lambda i: (i, 0)),
        out_shape=jax.ShapeDtypeStruct((m, n), x.dtype),
    )(x)
